# baseline (speedup 1.0000x reference)
import jax
import jax.numpy as jnp
from jax.experimental import pallas as pl
from jax.experimental.pallas import tpu as pltpu

_HIDDEN = 2048
_EXPERTS = 64
_BLK = 512
_NBUF = 4


def _router_pipeline(x_hbm, w_ref, o_hbm, b0, b1, b2, b3, o_vmem,
                     s0, s1, s2, s3, so):
    bufs = (b0, b1, b2, b3)
    sems = (s0, s1, s2, s3)
    n_chunks = x_hbm.shape[0] // _BLK

    def in_cp(k):
        return pltpu.make_async_copy(
            x_hbm.at[pl.ds(k * _BLK, _BLK), :], bufs[k % _NBUF], sems[k % _NBUF])

    for k in range(_NBUF):
        in_cp(k).start()

    w = w_ref[...].astype(jnp.bfloat16)
    dn = (((1,), (1,)), ((), ()))
    for i in range(n_chunks):
        in_cp(i).wait()
        x = bufs[i % _NBUF][...].astype(jnp.bfloat16)
        o_vmem[pl.ds(i * _BLK, _BLK), :] = jax.lax.dot_general(
            x, w, dn, preferred_element_type=jnp.float32)
        if i + _NBUF < n_chunks:
            in_cp(i + _NBUF).start()

    out_cp = pltpu.make_async_copy(o_vmem, o_hbm, so)
    out_cp.start()
    out_cp.wait()


def kernel(hidden_states, weight):
    hs = hidden_states.reshape(-1, _HIDDEN)
    n = hs.shape[0]
    return pl.pallas_call(
        _router_pipeline,
        in_specs=[
            pl.BlockSpec(memory_space=pltpu.HBM),
            pl.BlockSpec(memory_space=pltpu.VMEM),
        ],
        out_specs=pl.BlockSpec(memory_space=pltpu.HBM),
        out_shape=jax.ShapeDtypeStruct((n, _EXPERTS), jnp.float32),
        scratch_shapes=[
            pltpu.VMEM((_BLK, _HIDDEN), jnp.float32),
            pltpu.VMEM((_BLK, _HIDDEN), jnp.float32),
            pltpu.VMEM((_BLK, _HIDDEN), jnp.float32),
            pltpu.VMEM((_BLK, _HIDDEN), jnp.float32),
            pltpu.VMEM((16384, _EXPERTS), jnp.float32),
            pltpu.SemaphoreType.DMA,
            pltpu.SemaphoreType.DMA,
            pltpu.SemaphoreType.DMA,
            pltpu.SemaphoreType.DMA,
            pltpu.SemaphoreType.DMA,
        ],
        compiler_params=pltpu.CompilerParams(
            vmem_limit_bytes=100 * 1024 * 1024,
        ),
    )(hs, weight)


# speedup vs baseline: 1.0625x; 1.0625x over previous
import jax
import jax.numpy as jnp
from jax.experimental import pallas as pl
from jax.experimental.pallas import tpu as pltpu

_HIDDEN = 2048
_EXPERTS = 64
_BLK = 512
_NSPLIT = 2


def _router_block(*refs):
    xs = refs[:_NSPLIT]
    w_ref = refs[_NSPLIT]
    o_ref = refs[_NSPLIT + 1]
    w = w_ref[...].astype(jnp.bfloat16)
    dn = (((1,), (1,)), ((), ()))
    for s in range(_NSPLIT):
        x = xs[s][0].astype(jnp.bfloat16)
        o_ref[s] = jax.lax.dot_general(
            x, w, dn, preferred_element_type=jnp.float32)


def kernel(hidden_states, weight):
    hs = hidden_states.reshape(-1, _HIDDEN)
    n = hs.shape[0]
    rows = n // _NSPLIT
    hs3 = hs.reshape(_NSPLIT, rows, _HIDDEN)
    steps = rows // _BLK

    def x_spec(s):
        return pl.BlockSpec((1, _BLK, _HIDDEN), lambda i, s=s: (s, i, 0))

    out = pl.pallas_call(
        _router_block,
        grid=(steps,),
        in_specs=[x_spec(s) for s in range(_NSPLIT)] + [
            pl.BlockSpec((_EXPERTS, _HIDDEN), lambda i: (0, 0)),
        ],
        out_specs=pl.BlockSpec((_NSPLIT, _BLK, _EXPERTS), lambda i: (0, i, 0)),
        out_shape=jax.ShapeDtypeStruct((_NSPLIT, rows, _EXPERTS), jnp.float32),
        compiler_params=pltpu.CompilerParams(
            dimension_semantics=(pltpu.PARALLEL,),
            vmem_limit_bytes=100 * 1024 * 1024,
        ),
    )(*([hs3] * _NSPLIT), weight)
    return out.reshape(n, _EXPERTS)
